# Initial kernel scaffold; baseline (speedup 1.0000x reference)
import jax
import jax.numpy as jnp
from jax.experimental import pallas as pl


def kernel(x, edge_index, W_l1, W_r1, b1, W_l2, W_r2, b2):
    raise NotImplementedError("write your pallas kernel here")


# speedup vs baseline: 4.2008x; 4.2008x over previous
import functools

import jax
import jax.numpy as jnp
from jax import lax
from jax.experimental import pallas as pl
from jax.experimental.pallas import tpu as pltpu
from jax.experimental.pallas import tpu_sc as plsc

N = 10000
E = 320000
D = 128
H = 128

NC = 2
NS = 16
NW = NC * NS
CHUNK = 128
E_PAD = 327680
TOT_CHUNKS = E_PAD // CHUNK
CHUNKS_PER_TILE = E_PAD // (NW * CHUNK)
E_ALLOC = 335872
ACC_ROWS = 10112
ROWS_PER_TILE = ACC_ROWS // NS

BLK = 400
GRID = N // BLK


def _make_agg(n0, n1):
    assert n0 % 8 == 0 and n1 % 8 == 0 and min(n0, n1) >= 8

    def body(y_hbm, pk_hbm, zrows_hbm, out_hbm,
             pk_v, sr_v, dr_v, r0, r1, acc_sh,
             g0, g1, s0, s1, p0, p1, p2, p3):
        rows = (r0, r1)
        gsem = (g0, g1)
        ssem = (s0, s1)
        psem = (p0, p1, p2, p3)
        cid = lax.axis_index("c")
        sid = lax.axis_index("s")

        sl = pl.ds(sid * ROWS_PER_TILE, ROWS_PER_TILE)
        pltpu.sync_copy(zrows_hbm.at[sl], acc_sh.at[sl])

        n = jnp.where(cid == 0, n0, n1)
        base = jnp.where(cid == 0, sid * n0, NS * n0 + sid * n1)

        def issue_pk(j, s):
            pltpu.async_copy(pk_hbm.at[base + j], pk_v.at[s], psem[s])

        def wait_pk(s):
            pltpu.make_async_copy(
                pk_hbm.at[base], pk_v.at[s], psem[s]).wait()

        def unpack(j, s, b):
            for k in range(CHUNK // 16):
                p = pk_v[s, pl.ds(k * 16, 16)]
                sr_v[b, pl.ds(k * 16, 16)] = p & 16383
                dr_v[b, pl.ds(k * 16, 16)] = lax.shift_right_logical(p, 14)

        def issue_gather(b):
            pltpu.async_copy(y_hbm.at[sr_v.at[b]], rows[b], gsem[b])

        def wait_gather(b):
            pltpu.make_async_copy(
                y_hbm.at[sr_v.at[b]], rows[b], gsem[b]).wait()

        def issue_scatter(b):
            pltpu.async_copy(rows[b], acc_sh.at[dr_v.at[b]], ssem[b],
                             add=True)

        def wait_scatter(b):
            pltpu.make_async_copy(
                rows[b], acc_sh.at[dr_v.at[b]], ssem[b]).wait()

        for s in range(4):
            issue_pk(s, s)
        wait_pk(0)
        unpack(0, 0, 0)
        wait_pk(1)
        unpack(1, 1, 1)
        issue_gather(0)
        issue_gather(1)

        plsc.subcore_barrier()

        def main(jo, c):
            for k in range(4):
                j = jo * 4 + k
                b = k % 2
                wait_gather(b)
                issue_scatter(b)
                wait_scatter(b)
                wait_pk((k + 2) % 4)
                unpack(j + 2, (k + 2) % 4, b)
                issue_gather(b)
                issue_pk(j + 4, k)
            return c

        lax.fori_loop(0, (n - 4) // 4, main, 0)

        for k, s in ((0, 2), (1, 3)):
            b = k
            wait_gather(b)
            issue_scatter(b)
            wait_scatter(b)
            wait_pk(s)
            unpack(n - 2 + k, s, b)
            issue_gather(b)

        for b in range(2):
            wait_gather(b)
            issue_scatter(b)
            wait_scatter(b)

        plsc.subcore_barrier()

        pltpu.sync_copy(acc_sh.at[sl], out_hbm.at[cid, sl])

    return pl.kernel(
        body,
        out_type=jax.ShapeDtypeStruct((NC, ACC_ROWS, D), jnp.float32),
        mesh=plsc.VectorSubcoreMesh(core_axis_name="c", subcore_axis_name="s",
                                    num_cores=NC, num_subcores=NS),
        scratch_types=[
            pltpu.VMEM((4, CHUNK), jnp.int32),
            pltpu.VMEM((2, CHUNK), jnp.int32),
            pltpu.VMEM((2, CHUNK), jnp.int32),
            pltpu.VMEM((CHUNK, D), jnp.float32),
            pltpu.VMEM((CHUNK, D), jnp.float32),
            pltpu.VMEM_SHARED((ACC_ROWS, D), jnp.float32),
            pltpu.SemaphoreType.DMA,
            pltpu.SemaphoreType.DMA,
            pltpu.SemaphoreType.DMA,
            pltpu.SemaphoreType.DMA,
            pltpu.SemaphoreType.DMA,
            pltpu.SemaphoreType.DMA,
            pltpu.SemaphoreType.DMA,
            pltpu.SemaphoreType.DMA,
        ],
    )


CPT_FAST = 152
CPT_SLOW = 8


@functools.cache
def _get_agg():
    return _make_agg(CPT_FAST, CPT_SLOW)


def _agg(*args):
    return _get_agg()(*args)


def _deg_body(dst_hbm, zrows_hbm, ones_hbm, degout_hbm,
              dst_v, ones_v, deg_sh, dsem):
    cid = lax.axis_index("c")
    sid = lax.axis_index("s")
    wid = cid * NS + sid

    sl = pl.ds(sid * ROWS_PER_TILE, ROWS_PER_TILE)
    pltpu.sync_copy(zrows_hbm.at[sl], deg_sh.at[sl])
    pltpu.sync_copy(ones_hbm, ones_v)

    esl = pl.ds(wid * CHUNKS_PER_TILE, CHUNKS_PER_TILE)
    pltpu.sync_copy(dst_hbm.at[esl], dst_v)

    plsc.subcore_barrier()

    def step(j, carry):
        pltpu.async_copy(ones_v, deg_sh.at[dst_v.at[j]], dsem, add=True)
        return carry

    lax.fori_loop(0, CHUNKS_PER_TILE, step, 0)

    def drain(j, carry):
        pltpu.make_async_copy(ones_v, deg_sh.at[dst_v.at[0]], dsem).wait()
        return carry

    lax.fori_loop(0, CHUNKS_PER_TILE, drain, 0)

    plsc.subcore_barrier()

    pltpu.sync_copy(deg_sh.at[sl], degout_hbm.at[cid, sl])


@functools.cache
def _get_deg():
    return pl.kernel(
        _deg_body,
        out_type=jax.ShapeDtypeStruct((NC, ACC_ROWS, D), jnp.float32),
        mesh=plsc.VectorSubcoreMesh(core_axis_name="c", subcore_axis_name="s",
                                    num_cores=NC, num_subcores=NS),
        scratch_types=[
            pltpu.VMEM((CHUNKS_PER_TILE, CHUNK), jnp.int32),
            pltpu.VMEM((CHUNK, D), jnp.float32),
            pltpu.VMEM_SHARED((ACC_ROWS, D), jnp.float32),
            pltpu.SemaphoreType.DMA,
        ],
    )


def _deg(*args):
    return _get_deg()(*args)


def _mm2_body(x_ref, wl_ref, wr_ref, y_ref, r_ref):
    xb = x_ref[...]
    y_ref[...] = jnp.dot(xb, wl_ref[...], preferred_element_type=jnp.float32)
    r_ref[...] = jnp.dot(xb, wr_ref[...], preferred_element_type=jnp.float32)


def _mm2(x, wl, wr):
    return pl.pallas_call(
        _mm2_body,
        grid=(GRID,),
        in_specs=[
            pl.BlockSpec((BLK, D), lambda i: (i, 0)),
            pl.BlockSpec((D, H), lambda i: (0, 0)),
            pl.BlockSpec((D, H), lambda i: (0, 0)),
        ],
        out_specs=[
            pl.BlockSpec((BLK, H), lambda i: (i, 0)),
            pl.BlockSpec((BLK, H), lambda i: (i, 0)),
        ],
        out_shape=[
            jax.ShapeDtypeStruct((N, H), jnp.float32),
            jax.ShapeDtypeStruct((N, H), jnp.float32),
        ],
    )(x, wl, wr)


def _mid_body(p_ref, dp_ref, r1_ref, b1_ref, wl2_ref, h1_ref, x2_ref, y2_ref):
    s = p_ref[0] + p_ref[1]
    deg = dp_ref[0, :, :1] + dp_ref[1, :, :1]
    rd = 1.0 / jnp.maximum(deg, 1.0)
    h1 = s * rd + b1_ref[...] + r1_ref[...]
    x2 = jnp.maximum(h1, 0.0)
    h1_ref[...] = h1
    x2_ref[...] = x2
    y2_ref[...] = jnp.dot(x2, wl2_ref[...], preferred_element_type=jnp.float32)


def _mid(p, dp, r1, b1, wl2):
    return pl.pallas_call(
        _mid_body,
        grid=(GRID,),
        in_specs=[
            pl.BlockSpec((NC, BLK, H), lambda i: (0, i, 0)),
            pl.BlockSpec((NC, BLK, D), lambda i: (0, i, 0)),
            pl.BlockSpec((BLK, H), lambda i: (i, 0)),
            pl.BlockSpec((1, H), lambda i: (0, 0)),
            pl.BlockSpec((H, H), lambda i: (0, 0)),
        ],
        out_specs=[
            pl.BlockSpec((BLK, H), lambda i: (i, 0)),
            pl.BlockSpec((BLK, H), lambda i: (i, 0)),
            pl.BlockSpec((BLK, H), lambda i: (i, 0)),
        ],
        out_shape=[
            jax.ShapeDtypeStruct((N, H), jnp.float32),
            jax.ShapeDtypeStruct((N, H), jnp.float32),
            jax.ShapeDtypeStruct((N, H), jnp.float32),
        ],
    )(p, dp, r1, b1, wl2)


def _fin_body(p_ref, dp_ref, x2_ref, wr2_ref, b2_ref, h1_ref, out_ref):
    s = p_ref[0] + p_ref[1]
    deg = dp_ref[0, :, :1] + dp_ref[1, :, :1]
    rd = 1.0 / jnp.maximum(deg, 1.0)
    h2 = s * rd + b2_ref[...] + jnp.dot(
        x2_ref[...], wr2_ref[...], preferred_element_type=jnp.float32)
    out_ref[:, 0, :] = h1_ref[...]
    out_ref[:, 1, :] = h2


def _fin(p, dp, x2, wr2, b2, h1):
    return pl.pallas_call(
        _fin_body,
        grid=(GRID,),
        in_specs=[
            pl.BlockSpec((NC, BLK, H), lambda i: (0, i, 0)),
            pl.BlockSpec((NC, BLK, D), lambda i: (0, i, 0)),
            pl.BlockSpec((BLK, H), lambda i: (i, 0)),
            pl.BlockSpec((H, H), lambda i: (0, 0)),
            pl.BlockSpec((1, H), lambda i: (0, 0)),
            pl.BlockSpec((BLK, H), lambda i: (i, 0)),
        ],
        out_specs=pl.BlockSpec((BLK, 2, H), lambda i: (i, 0, 0)),
        out_shape=jax.ShapeDtypeStruct((N, 2, H), jnp.float32),
    )(p, dp, x2, wr2, b2, h1)


def kernel(x, edge_index, W_l1, W_r1, b1, W_l2, W_r2, b2):
    src = edge_index[0]
    dst = edge_index[1]
    pad = E_ALLOC - E
    srcp = jnp.concatenate([src, jnp.zeros((pad,), jnp.int32)])
    dstp = jnp.concatenate([dst, jnp.full((pad,), N, jnp.int32)])
    pk_p = ((dstp << 14) | srcp).reshape(-1, CHUNK)
    dst_p = dstp[:E_PAD].reshape(-1, CHUNK)

    zrows = jnp.zeros((ACC_ROWS, D), jnp.float32)
    ones = jnp.ones((CHUNK, D), jnp.float32)
    b1r = b1.reshape(1, H)
    b2r = b2.reshape(1, H)

    y1, r1 = _mm2(x, W_l1, W_r1)
    dp = _deg(dst_p, zrows, ones)
    p1 = _agg(y1, pk_p, zrows)
    h1, x2, y2 = _mid(p1, dp, r1, b1r, W_l2)
    p2 = _agg(y2, pk_p, zrows)
    return _fin(p2, dp, x2, W_r2, b2r, h1)
